# Initial kernel scaffold; baseline (speedup 1.0000x reference)
import jax
import jax.numpy as jnp
from jax.experimental import pallas as pl


def kernel(step_x, ei_implies, ei_semantic, ei_equiv, ei_contrib, step_proj_W, step_proj_b, ans_emb, ans_proj_W, ans_proj_b, conv1_Wl, conv1_bl, conv1_Wr, conv2_Wl, conv2_bl, conv2_Wr, pred_W, pred_b):
    raise NotImplementedError("write your pallas kernel here")


# speedup vs baseline: 1.0047x; 1.0047x over previous
import functools

import jax
import jax.numpy as jnp
from jax.experimental import pallas as pl
from jax.experimental.pallas import tpu as pltpu

H = 512


def _linear_body(a_ref, w_ref, b_ref, o_ref):
    acc = jnp.dot(a_ref[...], w_ref[...], preferred_element_type=jnp.float32)
    o_ref[...] = acc + b_ref[...]


def tc_linear(a, w, b, bm):
    m, k = a.shape
    n = w.shape[1]
    grid = (m // bm,)
    return pl.pallas_call(
        _linear_body,
        grid=grid,
        in_specs=[
            pl.BlockSpec((bm, k), lambda i: (i, 0)),
            pl.BlockSpec((k, n), lambda i: (0, 0)),
            pl.BlockSpec((1, n), lambda i: (0, 0)),
        ],
        out_specs=pl.BlockSpec((bm, n), lambda i: (i, 0)),
        out_shape=jax.ShapeDtypeStruct((m, n), jnp.float32),
    )(a, w, b.reshape(1, n))


def _sage_body(a1_ref, i1_ref, a2_ref, i2_ref, x_ref, w_ref, b_ref, o_ref):
    m1 = a1_ref[...] * i1_ref[...]
    m2 = a2_ref[...] * i2_ref[...]
    acc = jnp.dot(m1, w_ref[0], preferred_element_type=jnp.float32)
    acc += jnp.dot(m2, w_ref[1], preferred_element_type=jnp.float32)
    acc += jnp.dot(x_ref[...], w_ref[2], preferred_element_type=jnp.float32)
    o_ref[...] = jnp.maximum(acc + b_ref[...], 0.0)


def tc_sage(a1, inv1, a2, inv2, x, w3, bias, bm):
    m = a1.shape[0]
    grid = (m // bm,)
    return pl.pallas_call(
        _sage_body,
        grid=grid,
        in_specs=[
            pl.BlockSpec((bm, H), lambda i: (i, 0)),
            pl.BlockSpec((bm, 1), lambda i: (i, 0)),
            pl.BlockSpec((bm, H), lambda i: (i, 0)),
            pl.BlockSpec((bm, 1), lambda i: (i, 0)),
            pl.BlockSpec((bm, H), lambda i: (i, 0)),
            pl.BlockSpec((3, H, H), lambda i: (0, 0, 0)),
            pl.BlockSpec((1, H), lambda i: (0, 0)),
        ],
        out_specs=pl.BlockSpec((bm, H), lambda i: (i, 0)),
        out_shape=jax.ShapeDtypeStruct((m, H), jnp.float32),
    )(a1, inv1, a2, inv2, x, w3, bias.reshape(1, H))


def _sage_pred_body(a1_ref, i1_ref, a2_ref, i2_ref, x_ref, w_ref, b_ref,
                    pw_ref, pb_ref, o_ref):
    m1 = a1_ref[...] * i1_ref[...]
    m2 = a2_ref[...] * i2_ref[...]
    acc = jnp.dot(m1, w_ref[0], preferred_element_type=jnp.float32)
    acc += jnp.dot(m2, w_ref[1], preferred_element_type=jnp.float32)
    acc += jnp.dot(x_ref[...], w_ref[2], preferred_element_type=jnp.float32)
    h = jnp.maximum(acc + b_ref[...], 0.0)
    logits = jnp.sum(h * pw_ref[...], axis=1) + pb_ref[0, 0]
    o_ref[...] = jnp.broadcast_to(jax.nn.sigmoid(logits)[:, None],
                                  o_ref.shape)


def tc_sage_pred(a1, inv1, a2, inv2, x, w3, bias, pred_w, pred_b, bm):
    m = a1.shape[0]
    grid = (m // bm,)
    return pl.pallas_call(
        _sage_pred_body,
        grid=grid,
        in_specs=[
            pl.BlockSpec((bm, H), lambda i: (i, 0)),
            pl.BlockSpec((bm, 1), lambda i: (i, 0)),
            pl.BlockSpec((bm, H), lambda i: (i, 0)),
            pl.BlockSpec((bm, 1), lambda i: (i, 0)),
            pl.BlockSpec((bm, H), lambda i: (i, 0)),
            pl.BlockSpec((3, H, H), lambda i: (0, 0, 0)),
            pl.BlockSpec((1, H), lambda i: (0, 0)),
            pl.BlockSpec((1, H), lambda i: (0, 0)),
            pl.BlockSpec((1, 1), lambda i: (0, 0), memory_space=pltpu.SMEM),
        ],
        out_specs=pl.BlockSpec((bm, 128), lambda i: (i, 0)),
        out_shape=jax.ShapeDtypeStruct((m, 128), jnp.float32),
    )(a1, inv1, a2, inv2, x, w3, bias.reshape(1, H),
      pred_w.reshape(1, H), pred_b.reshape(1, 1))[:, 0]


def _segsum(x_src, src, dst, n_dst):
    msgs = jnp.take(x_src, src, axis=0)
    return jax.ops.segment_sum(msgs, dst, num_segments=n_dst)


def _counts(dst, n_dst):
    return jax.ops.segment_sum(jnp.ones(dst.shape, jnp.float32), dst,
                               num_segments=n_dst)


def kernel(step_x, ei_implies, ei_semantic, ei_equiv, ei_contrib,
           step_proj_W, step_proj_b, ans_emb, ans_proj_W, ans_proj_b,
           conv1_Wl, conv1_bl, conv1_Wr, conv2_Wl, conv2_bl, conv2_Wr,
           pred_W, pred_b):
    n_step = step_x.shape[0]
    n_ans = ans_emb.shape[0]

    x_step = tc_linear(step_x, step_proj_W, step_proj_b, bm=1000)
    x_ans = tc_linear(ans_emb, ans_proj_W, ans_proj_b, bm=1000)

    inv_imp = (1.0 / jnp.clip(_counts(ei_implies[1], n_step), 1.0, None)).reshape(-1, 1)
    inv_sem = (1.0 / jnp.clip(_counts(ei_semantic[1], n_step), 1.0, None)).reshape(-1, 1)
    inv_eqv = (1.0 / jnp.clip(_counts(ei_equiv[1], n_ans), 1.0, None)).reshape(-1, 1)
    inv_ctr = (1.0 / jnp.clip(_counts(ei_contrib[1], n_ans), 1.0, None)).reshape(-1, 1)

    w_step = [jnp.stack([cWl[0], cWl[1], cWr[0] + cWr[1]])
              for cWl, cWr in ((conv1_Wl, conv1_Wr), (conv2_Wl, conv2_Wr))]
    b_step = [conv1_bl[0] + conv1_bl[1], conv2_bl[0] + conv2_bl[1]]
    w_ans = [jnp.stack([cWl[2], cWl[3], cWr[2] + cWr[3]])
             for cWl, cWr in ((conv1_Wl, conv1_Wr), (conv2_Wl, conv2_Wr))]
    b_ans = [conv1_bl[2] + conv1_bl[3], conv2_bl[2] + conv2_bl[3]]

    for layer in range(6):
        p = layer % 2
        agg_eqv = _segsum(x_ans, ei_equiv[0], ei_equiv[1], n_ans)
        agg_ctr = _segsum(x_step, ei_contrib[0], ei_contrib[1], n_ans)
        if layer < 5:
            agg_imp = _segsum(x_step, ei_implies[0], ei_implies[1], n_step)
            agg_sem = _segsum(x_step, ei_semantic[0], ei_semantic[1], n_step)
            new_step = tc_sage(agg_imp, inv_imp, agg_sem, inv_sem, x_step,
                               w_step[p], b_step[p], bm=1000)
            x_ans = tc_sage(agg_eqv, inv_eqv, agg_ctr, inv_ctr, x_ans,
                            w_ans[p], b_ans[p], bm=1000)
            x_step = new_step
        else:
            return tc_sage_pred(agg_eqv, inv_eqv, agg_ctr, inv_ctr, x_ans,
                                w_ans[p], b_ans[p], pred_W[:, 0], pred_b,
                                bm=1000)
